# baseline (speedup 1.0000x reference)
import jax
import jax.numpy as jnp
from jax.experimental import pallas as pl

_INV_T = 20.0
_TILE_N = 512


def _mm_kernel(x_ref, m_ref, o_ref):
    a = x_ref[...] * _INV_T
    o_ref[...] = jax.lax.dot_general(
        a, m_ref[...],
        dimension_numbers=(((1,), (1,)), ((), ())),
        preferred_element_type=jnp.float32)


def kernel(x, y, memory):
    del y
    b, k = x.shape
    n = memory.shape[0]
    return pl.pallas_call(
        _mm_kernel,
        grid=(pl.cdiv(n, _TILE_N),),
        in_specs=[
            pl.BlockSpec((b, k), lambda i: (0, 0)),
            pl.BlockSpec((_TILE_N, k), lambda i: (i, 0)),
        ],
        out_specs=pl.BlockSpec((b, _TILE_N), lambda i: (0, i)),
        out_shape=jax.ShapeDtypeStruct((b, n), jnp.float32),
    )(x, memory)


# speedup vs baseline: 1.0423x; 1.0423x over previous
import functools

import jax
import jax.numpy as jnp
from jax.experimental import pallas as pl
from jax.experimental.pallas import tpu as pltpu

_INV_T = 20.0
_TILE_N = 512
_NBUF = 8
_TAIL_BLK = 2048


def _tail_kernel(x_ref, m_ref, o_ref):
    a = x_ref[...] * _INV_T
    o_ref[...] = jax.lax.dot_general(
        a, m_ref[...],
        dimension_numbers=(((1,), (1,)), ((), ())),
        preferred_element_type=jnp.float32)


def _main_kernel(x_ref, m_ref, o_alias, o_hbm, acc_ref, sems, *, nt):
    del o_alias
    i = pl.program_id(0)
    slot = jax.lax.rem(i, _NBUF)

    @pl.when(i >= _NBUF)
    def _wait_prev():
        pltpu.make_async_copy(
            acc_ref.at[slot],
            o_hbm.at[:, pl.ds((i - _NBUF) * _TILE_N, _TILE_N)],
            sems.at[slot],
        ).wait()

    a = x_ref[...] * _INV_T
    acc_ref[slot] = jax.lax.dot_general(
        a, m_ref[...],
        dimension_numbers=(((1,), (1,)), ((), ())),
        preferred_element_type=jnp.float32)
    pltpu.make_async_copy(
        acc_ref.at[slot],
        o_hbm.at[:, pl.ds(i * _TILE_N, _TILE_N)],
        sems.at[slot],
    ).start()

    @pl.when(i == nt - 1)
    def _drain():
        for d in range(min(_NBUF, nt)):
            step = nt - 1 - d
            s = step % _NBUF
            pltpu.make_async_copy(
                acc_ref.at[s],
                o_hbm.at[:, pl.ds(step * _TILE_N, _TILE_N)],
                sems.at[s],
            ).wait()


def kernel(x, y, memory):
    del y
    b, k = x.shape
    n = memory.shape[0]
    last_blk = (n - 1) // _TAIL_BLK
    n_main = last_blk * _TAIL_BLK
    nt = n_main // _TILE_N

    out0 = pl.pallas_call(
        _tail_kernel,
        grid=(1,),
        in_specs=[
            pl.BlockSpec((b, k), lambda i: (0, 0)),
            pl.BlockSpec((_TAIL_BLK, k), lambda i: (last_blk, 0)),
        ],
        out_specs=pl.BlockSpec((b, _TAIL_BLK), lambda i: (0, last_blk)),
        out_shape=jax.ShapeDtypeStruct((b, n), jnp.float32),
    )(x, memory)

    return pl.pallas_call(
        functools.partial(_main_kernel, nt=nt),
        grid=(nt,),
        in_specs=[
            pl.BlockSpec((b, k), lambda i: (0, 0)),
            pl.BlockSpec((_TILE_N, k), lambda i: (i, 0)),
            pl.BlockSpec(memory_space=pltpu.MemorySpace.HBM),
        ],
        out_specs=pl.BlockSpec(memory_space=pltpu.MemorySpace.HBM),
        out_shape=jax.ShapeDtypeStruct((b, n), jnp.float32),
        input_output_aliases={2: 0},
        scratch_shapes=[
            pltpu.VMEM((_NBUF, b, _TILE_N), jnp.float32),
            pltpu.SemaphoreType.DMA((_NBUF,)),
        ],
    )(x, memory, out0)


# speedup vs baseline: 1.2645x; 1.2132x over previous
import functools

import jax
import jax.numpy as jnp
from jax.experimental import pallas as pl
from jax.experimental.pallas import tpu as pltpu

_INV_T = 20.0
_TILE_M = 16
_NBUF = 5


def _mm_kernel(x_ref, mt_ref, o_hbm, acc_ref, sems, *, nt):
    i = pl.program_id(0)
    slot = jax.lax.rem(i, _NBUF)

    @pl.when(i >= _NBUF)
    def _wait_prev():
        pltpu.make_async_copy(
            acc_ref.at[slot],
            o_hbm.at[pl.ds((i - _NBUF) * _TILE_M, _TILE_M), :],
            sems.at[slot],
        ).wait()

    a = x_ref[...] * _INV_T
    acc_ref[slot] = jax.lax.dot_general(
        a, mt_ref[...],
        dimension_numbers=(((1,), (0,)), ((), ())),
        preferred_element_type=jnp.float32)
    pltpu.make_async_copy(
        acc_ref.at[slot],
        o_hbm.at[pl.ds(i * _TILE_M, _TILE_M), :],
        sems.at[slot],
    ).start()

    @pl.when(i == nt - 1)
    def _drain():
        for d in range(min(_NBUF, nt)):
            step = nt - 1 - d
            s = step % _NBUF
            pltpu.make_async_copy(
                acc_ref.at[s],
                o_hbm.at[pl.ds(step * _TILE_M, _TILE_M), :],
                sems.at[s],
            ).wait()


def kernel(x, y, memory):
    del y
    b, k = x.shape
    n = memory.shape[0]
    nt = b // _TILE_M
    mem_t = memory.T
    return pl.pallas_call(
        functools.partial(_mm_kernel, nt=nt),
        grid=(nt,),
        in_specs=[
            pl.BlockSpec((_TILE_M, k), lambda i: (i, 0)),
            pl.BlockSpec((k, n), lambda i: (0, 0)),
        ],
        out_specs=pl.BlockSpec(memory_space=pltpu.MemorySpace.HBM),
        out_shape=jax.ShapeDtypeStruct((b, n), jnp.float32),
        scratch_shapes=[
            pltpu.VMEM((_NBUF, _TILE_M, n), jnp.float32),
            pltpu.SemaphoreType.DMA((_NBUF,)),
        ],
        compiler_params=pltpu.CompilerParams(
            vmem_limit_bytes=63 * 1024 * 1024,
        ),
    )(x, mem_t)


# speedup vs baseline: 1.2661x; 1.0013x over previous
import functools

import jax
import jax.numpy as jnp
from jax.experimental import pallas as pl
from jax.experimental.pallas import tpu as pltpu

_INV_T = 20.0
_TILE_M = 16
_NBUF = 5


def _mm_kernel(x_ref, mt_ref, o_hbm, acc_ref, sems, *, nt):
    i = pl.program_id(0)
    slot = jax.lax.rem(i, _NBUF)

    for s in range(_NBUF):
        @pl.when(jnp.logical_and(i >= _NBUF, slot == s))
        def _wait_prev(s=s):
            pltpu.make_async_copy(
                acc_ref.at[s],
                o_hbm.at[pl.ds((i - _NBUF) * _TILE_M, _TILE_M), :],
                sems.at[s],
            ).wait()

    a = x_ref[...] * _INV_T
    acc_ref[slot] = jax.lax.dot_general(
        a, mt_ref[...],
        dimension_numbers=(((1,), (0,)), ((), ())),
        preferred_element_type=jnp.float32)
    for s in range(_NBUF):
        @pl.when(slot == s)
        def _start(s=s):
            pltpu.make_async_copy(
                acc_ref.at[s],
                o_hbm.at[pl.ds(i * _TILE_M, _TILE_M), :],
                sems.at[s],
            ).start()

    @pl.when(i == nt - 1)
    def _drain():
        for d in range(min(_NBUF, nt)):
            step = nt - 1 - d
            s = step % _NBUF
            pltpu.make_async_copy(
                acc_ref.at[s],
                o_hbm.at[pl.ds(step * _TILE_M, _TILE_M), :],
                sems.at[s],
            ).wait()


def kernel(x, y, memory):
    del y
    b, k = x.shape
    n = memory.shape[0]
    nt = b // _TILE_M
    mem_t = memory.T
    return pl.pallas_call(
        functools.partial(_mm_kernel, nt=nt),
        grid=(nt,),
        in_specs=[
            pl.BlockSpec((_TILE_M, k), lambda i: (i, 0)),
            pl.BlockSpec((k, n), lambda i: (0, 0)),
        ],
        out_specs=pl.BlockSpec(memory_space=pltpu.MemorySpace.HBM),
        out_shape=jax.ShapeDtypeStruct((b, n), jnp.float32),
        scratch_shapes=[
            pltpu.VMEM((_NBUF, _TILE_M, n), jnp.float32),
            pltpu.SemaphoreType.DMA((_NBUF,)),
        ],
        compiler_params=pltpu.CompilerParams(
            vmem_limit_bytes=63 * 1024 * 1024,
        ),
    )(x, mem_t)


# speedup vs baseline: 3.2657x; 2.5793x over previous
import functools

import jax
import jax.numpy as jnp
from jax.experimental import pallas as pl
from jax.experimental.pallas import tpu as pltpu

_INV_T = 20.0
_TILE_R = 1000
_NBUF = 8


def _mm_kernel(m_ref, x_ref, o_hbm, acc_ref, sems, *, nt):
    i = pl.program_id(0)
    slot = jax.lax.rem(i, _NBUF)

    @pl.when(i >= _NBUF)
    def _wait_prev():
        pltpu.make_async_copy(
            acc_ref.at[slot],
            o_hbm.at[pl.ds((i - _NBUF) * _TILE_R, _TILE_R), :],
            sems.at[slot],
        ).wait()

    a = m_ref[...] * _INV_T
    acc_ref[slot] = jax.lax.dot_general(
        a, x_ref[...],
        dimension_numbers=(((1,), (1,)), ((), ())),
        preferred_element_type=jnp.float32)
    pltpu.make_async_copy(
        acc_ref.at[slot],
        o_hbm.at[pl.ds(i * _TILE_R, _TILE_R), :],
        sems.at[slot],
    ).start()

    @pl.when(i == nt - 1)
    def _drain():
        for d in range(min(_NBUF, nt)):
            step = nt - 1 - d
            s = step % _NBUF
            pltpu.make_async_copy(
                acc_ref.at[s],
                o_hbm.at[pl.ds(step * _TILE_R, _TILE_R), :],
                sems.at[s],
            ).wait()


def kernel(x, y, memory):
    del y
    b, k = x.shape
    n = memory.shape[0]
    nt = n // _TILE_R
    out_t = pl.pallas_call(
        functools.partial(_mm_kernel, nt=nt),
        grid=(nt,),
        in_specs=[
            pl.BlockSpec((_TILE_R, k), lambda i: (i, 0)),
            pl.BlockSpec((b, k), lambda i: (0, 0)),
        ],
        out_specs=pl.BlockSpec(memory_space=pltpu.MemorySpace.HBM),
        out_shape=jax.ShapeDtypeStruct((n, b), jnp.float32),
        scratch_shapes=[
            pltpu.VMEM((_NBUF, _TILE_R, b), jnp.float32),
            pltpu.SemaphoreType.DMA((_NBUF,)),
        ],
        compiler_params=pltpu.CompilerParams(
            vmem_limit_bytes=63 * 1024 * 1024,
        ),
    )(memory, x)
    return out_t.T


# speedup vs baseline: 3.2665x; 1.0003x over previous
import functools

import jax
import jax.numpy as jnp
from jax.experimental import pallas as pl
from jax.experimental.pallas import tpu as pltpu

_INV_T = 20.0
_TILE_R = 1000
_NBUF = 12


def _mm_kernel(m_ref, x_ref, o_hbm, acc_ref, sems, *, nt):
    i = pl.program_id(0)
    slot = jax.lax.rem(i, _NBUF)

    @pl.when(i >= _NBUF)
    def _wait_prev():
        pltpu.make_async_copy(
            acc_ref.at[slot],
            o_hbm.at[pl.ds((i - _NBUF) * _TILE_R, _TILE_R), :],
            sems.at[slot],
        ).wait()

    a = m_ref[...] * _INV_T
    acc_ref[slot] = jax.lax.dot_general(
        a, x_ref[...],
        dimension_numbers=(((1,), (1,)), ((), ())),
        preferred_element_type=jnp.float32)
    pltpu.make_async_copy(
        acc_ref.at[slot],
        o_hbm.at[pl.ds(i * _TILE_R, _TILE_R), :],
        sems.at[slot],
    ).start()

    @pl.when(i == nt - 1)
    def _drain():
        for d in range(min(_NBUF, nt)):
            step = nt - 1 - d
            s = step % _NBUF
            pltpu.make_async_copy(
                acc_ref.at[s],
                o_hbm.at[pl.ds(step * _TILE_R, _TILE_R), :],
                sems.at[s],
            ).wait()


def kernel(x, y, memory):
    del y
    b, k = x.shape
    n = memory.shape[0]
    nt = n // _TILE_R
    out_t = pl.pallas_call(
        functools.partial(_mm_kernel, nt=nt),
        grid=(nt,),
        in_specs=[
            pl.BlockSpec((_TILE_R, k), lambda i: (i, 0)),
            pl.BlockSpec((b, k), lambda i: (0, 0)),
        ],
        out_specs=pl.BlockSpec(memory_space=pltpu.MemorySpace.HBM),
        out_shape=jax.ShapeDtypeStruct((n, b), jnp.float32),
        scratch_shapes=[
            pltpu.VMEM((_NBUF, _TILE_R, b), jnp.float32),
            pltpu.SemaphoreType.DMA((_NBUF,)),
        ],
        compiler_params=pltpu.CompilerParams(
            vmem_limit_bytes=63 * 1024 * 1024,
        ),
    )(memory, x)
    return out_t.T


# speedup vs baseline: 3.4107x; 1.0441x over previous
import functools

import jax
import jax.numpy as jnp
from jax.experimental import pallas as pl
from jax.experimental.pallas import tpu as pltpu

_INV_T = 20.0
_TILE_R = 2000
_NBUF = 6


def _mm_kernel(m_ref, x_ref, o_hbm, acc_ref, sems, *, nt):
    i = pl.program_id(0)
    slot = jax.lax.rem(i, _NBUF)

    @pl.when(i >= _NBUF)
    def _wait_prev():
        pltpu.make_async_copy(
            acc_ref.at[slot],
            o_hbm.at[pl.ds((i - _NBUF) * _TILE_R, _TILE_R), :],
            sems.at[slot],
        ).wait()

    a = m_ref[...] * _INV_T
    acc_ref[slot] = jax.lax.dot_general(
        a, x_ref[...],
        dimension_numbers=(((1,), (1,)), ((), ())),
        preferred_element_type=jnp.float32)
    pltpu.make_async_copy(
        acc_ref.at[slot],
        o_hbm.at[pl.ds(i * _TILE_R, _TILE_R), :],
        sems.at[slot],
    ).start()

    @pl.when(i == nt - 1)
    def _drain():
        for d in range(min(_NBUF, nt)):
            step = nt - 1 - d
            s = step % _NBUF
            pltpu.make_async_copy(
                acc_ref.at[s],
                o_hbm.at[pl.ds(step * _TILE_R, _TILE_R), :],
                sems.at[s],
            ).wait()


def kernel(x, y, memory):
    del y
    b, k = x.shape
    n = memory.shape[0]
    nt = n // _TILE_R
    out_t = pl.pallas_call(
        functools.partial(_mm_kernel, nt=nt),
        grid=(nt,),
        in_specs=[
            pl.BlockSpec((_TILE_R, k), lambda i: (i, 0)),
            pl.BlockSpec((b, k), lambda i: (0, 0)),
        ],
        out_specs=pl.BlockSpec(memory_space=pltpu.MemorySpace.HBM),
        out_shape=jax.ShapeDtypeStruct((n, b), jnp.float32),
        scratch_shapes=[
            pltpu.VMEM((_NBUF, _TILE_R, b), jnp.float32),
            pltpu.SemaphoreType.DMA((_NBUF,)),
        ],
        compiler_params=pltpu.CompilerParams(
            vmem_limit_bytes=63 * 1024 * 1024,
        ),
    )(memory, x)
    return out_t.T
